# baseline (speedup 1.0000x reference)
import functools

import jax
import jax.numpy as jnp
from jax import lax
from jax.experimental import pallas as pl
from jax.experimental.pallas import tpu as pltpu
from jax.experimental.pallas import tpu_sc as plsc

_D = 128
_CHUNK = 128
_NBUF = 2
_LANES = 16
_GROUPS = _D // _LANES


@functools.lru_cache(maxsize=None)
def _make_partial_kernel(nw: int, nc: int, chunks: int):
  mesh = plsc.VectorSubcoreMesh(core_axis_name="c", subcore_axis_name="s")

  row_bufs = [pltpu.VMEM((3 * _CHUNK, _D), jnp.float32)
              for _ in range(_NBUF)]

  @functools.partial(
      pl.kernel,
      mesh=mesh,
      out_type=jax.ShapeDtypeStruct((nw, _LANES), jnp.float32),
      scratch_types=[
          pltpu.VMEM((chunks * 3 * _CHUNK,), jnp.int32),
          *row_bufs,
          pltpu.VMEM((_LANES,), jnp.float32),
          *[pltpu.SemaphoreType.DMA for _ in range(2 * _NBUF + 1)],
      ],
  )
  def partial_kernel(idx_hbm, emb_hbm, out_hbm, idx, *rest):
    rows = rest[:_NBUF]
    accv = rest[_NBUF]
    sems = rest[_NBUF + 1:]
    sem_idx = sems[2 * _NBUF]

    wid = lax.axis_index("s") * nc + lax.axis_index("c")
    pltpu.async_copy(idx_hbm.at[wid], idx, sem_idx).wait()

    half = 3 * _CHUNK // 2

    def fire(j):
      k = j % _NBUF
      buf = rows[k]
      return (
          pltpu.async_copy(
              emb_hbm.at[idx.at[pl.ds(3 * _CHUNK * j, half)]],
              buf.at[pl.ds(0, half)], sems[2 * k]),
          pltpu.async_copy(
              emb_hbm.at[idx.at[pl.ds(3 * _CHUNK * j + half, half)]],
              buf.at[pl.ds(half, half)], sems[2 * k + 1]),
      )

    inflight = {0: fire(0)}
    acc = tuple(jnp.zeros((_LANES,), jnp.float32) for _ in range(2 * _GROUPS))
    for j in range(chunks):
      if j + 1 < chunks:
        inflight[j + 1] = fire(j + 1)
      for cp in inflight.pop(j):
        cp.wait()
      buf = rows[j % _NBUF]

      def row_body(i, a, buf=buf):
        r0 = 2 * i
        r1 = r0 + 1
        new = list(a)
        for g in range(_GROUPS):
          sl = pl.ds(g * _LANES, _LANES)
          new[g] = new[g] + buf[r0, sl] * (
              buf[_CHUNK + r0, sl] - buf[2 * _CHUNK + r0, sl])
          new[_GROUPS + g] = new[_GROUPS + g] + buf[r1, sl] * (
              buf[_CHUNK + r1, sl] - buf[2 * _CHUNK + r1, sl])
        return tuple(new)

      acc = lax.fori_loop(0, _CHUNK // 2, row_body, acc)

    total = acc[0]
    for g in range(1, 2 * _GROUPS):
      total = total + acc[g]
    accv[...] = total
    pltpu.sync_copy(accv, out_hbm.at[wid])

  return partial_kernel


def kernel(target, context, negative_samples, embeddings):
  b = target.shape[0]
  info = plsc.get_sparse_core_info()
  nw = info.num_cores * info.num_subcores
  per_w = b // nw
  chunks = per_w // _CHUNK
  t = target.reshape(nw, chunks, 1, _CHUNK)
  c = context.reshape(nw, chunks, 1, _CHUNK)
  n = negative_samples.reshape(nw, chunks, 1, _CHUNK)
  comb = jnp.concatenate([t, c, n], axis=2).reshape(nw, chunks * 3 * _CHUNK)
  partials = _make_partial_kernel(nw, info.num_cores, chunks)(
      comb, embeddings)
  return -(jnp.sum(partials) / b)


# speedup vs baseline: 1.0093x; 1.0093x over previous
import functools

import jax
import jax.numpy as jnp
from jax import lax
from jax.experimental import pallas as pl
from jax.experimental.pallas import tpu as pltpu
from jax.experimental.pallas import tpu_sc as plsc

_D = 128
_CHUNK = 128
_NBUF = 2
_LANES = 16
_GROUPS = _D // _LANES


@functools.lru_cache(maxsize=None)
def _make_partial_kernel(nw: int, nc: int, chunks: int):
  mesh = plsc.VectorSubcoreMesh(core_axis_name="c", subcore_axis_name="s")

  row_bufs = [pltpu.VMEM((3 * _CHUNK, _D), jnp.float32)
              for _ in range(_NBUF)]

  @functools.partial(
      pl.kernel,
      mesh=mesh,
      out_type=jax.ShapeDtypeStruct((nw, _LANES), jnp.float32),
      scratch_types=[
          pltpu.VMEM((chunks * 3 * _CHUNK,), jnp.int32),
          *row_bufs,
          pltpu.VMEM((_LANES,), jnp.float32),
          *[pltpu.SemaphoreType.DMA for _ in range(2 * _NBUF + 1)],
      ],
  )
  def partial_kernel(idx_hbm, emb_hbm, out_hbm, idx, *rest):
    rows = rest[:_NBUF]
    accv = rest[_NBUF]
    sems = rest[_NBUF + 1:]
    sem_idx = sems[2 * _NBUF]

    wid = lax.axis_index("s") * nc + lax.axis_index("c")
    pltpu.async_copy(idx_hbm.at[wid], idx, sem_idx).wait()

    def fire(j):
      return pltpu.async_copy(emb_hbm.at[idx.at[pl.ds(3 * _CHUNK * j,
                                                      3 * _CHUNK)]],
                              rows[j % _NBUF], sems[j % _NBUF])

    inflight = {0: fire(0)}
    acc = tuple(jnp.zeros((_LANES,), jnp.float32) for _ in range(2 * _GROUPS))
    for j in range(chunks):
      if j + 1 < chunks:
        inflight[j + 1] = fire(j + 1)
      inflight.pop(j).wait()
      buf = rows[j % _NBUF]

      def row_body(i, a, buf=buf):
        r0 = 2 * i
        r1 = r0 + 1
        new = list(a)
        for g in range(_GROUPS):
          sl = pl.ds(g * _LANES, _LANES)
          new[g] = new[g] + buf[r0, sl] * (
              buf[_CHUNK + r0, sl] - buf[2 * _CHUNK + r0, sl])
          new[_GROUPS + g] = new[_GROUPS + g] + buf[r1, sl] * (
              buf[_CHUNK + r1, sl] - buf[2 * _CHUNK + r1, sl])
        return tuple(new)

      acc = lax.fori_loop(0, _CHUNK // 2, row_body, acc)

    total = acc[0]
    for g in range(1, 2 * _GROUPS):
      total = total + acc[g]
    accv[...] = total
    pltpu.sync_copy(accv, out_hbm.at[wid])

  return partial_kernel


def kernel(target, context, negative_samples, embeddings):
  b = target.shape[0]
  info = plsc.get_sparse_core_info()
  nw = info.num_cores * info.num_subcores
  per_w = b // nw
  chunks = per_w // _CHUNK
  t = target.reshape(nw, chunks, 1, _CHUNK)
  c = context.reshape(nw, chunks, 1, _CHUNK)
  n = negative_samples.reshape(nw, chunks, 1, _CHUNK)
  comb = jnp.concatenate([t, c, n], axis=2).reshape(nw, chunks * 3 * _CHUNK)
  partials = _make_partial_kernel(nw, info.num_cores, chunks)(
      comb, embeddings)
  return -(jnp.sum(partials) / b)


# speedup vs baseline: 1.0101x; 1.0008x over previous
import functools

import jax
import jax.numpy as jnp
from jax import lax
from jax.experimental import pallas as pl
from jax.experimental.pallas import tpu as pltpu
from jax.experimental.pallas import tpu_sc as plsc

_D = 128
_CHUNK = 128
_NBUF = 2
_LANES = 16
_GROUPS = _D // _LANES


@functools.lru_cache(maxsize=None)
def _make_partial_kernel(nw: int, nc: int, chunks: int):
  mesh = plsc.VectorSubcoreMesh(core_axis_name="c", subcore_axis_name="s")

  row_bufs = [pltpu.VMEM((3 * _CHUNK, _D), jnp.float32)
              for _ in range(_NBUF)]

  @functools.partial(
      pl.kernel,
      mesh=mesh,
      out_type=jax.ShapeDtypeStruct((nw, _LANES), jnp.float32),
      scratch_types=[
          pltpu.VMEM((chunks * 3 * _CHUNK,), jnp.int32),
          *row_bufs,
          pltpu.VMEM((_LANES,), jnp.float32),
          *[pltpu.SemaphoreType.DMA for _ in range(2 * _NBUF + 1)],
      ],
  )
  def partial_kernel(idx_hbm, emb_hbm, out_hbm, idx, *rest):
    rows = rest[:_NBUF]
    accv = rest[_NBUF]
    sems = rest[_NBUF + 1:]
    sem_idx0 = sems[2 * _NBUF - 1]
    sem_idx = sems[2 * _NBUF]

    wid = lax.axis_index("s") * nc + lax.axis_index("c")
    per = 3 * _CHUNK
    cp_idx0 = pltpu.async_copy(idx_hbm.at[wid, pl.ds(0, per)],
                               idx.at[pl.ds(0, per)], sem_idx0)
    cp_rest = pltpu.async_copy(
        idx_hbm.at[wid, pl.ds(per, (chunks - 1) * per)],
        idx.at[pl.ds(per, (chunks - 1) * per)], sem_idx)

    def fire(j):
      return pltpu.async_copy(emb_hbm.at[idx.at[pl.ds(3 * _CHUNK * j,
                                                      3 * _CHUNK)]],
                              rows[j % _NBUF], sems[j % _NBUF])

    cp_idx0.wait()
    inflight = {0: fire(0)}
    cp_rest.wait()
    acc = tuple(jnp.zeros((_LANES,), jnp.float32) for _ in range(2 * _GROUPS))
    for j in range(chunks):
      if j + 1 < chunks:
        inflight[j + 1] = fire(j + 1)
      inflight.pop(j).wait()
      buf = rows[j % _NBUF]

      def row_body(i, a, buf=buf):
        r0 = 2 * i
        r1 = r0 + 1
        new = list(a)
        for g in range(_GROUPS):
          sl = pl.ds(g * _LANES, _LANES)
          new[g] = new[g] + buf[r0, sl] * (
              buf[_CHUNK + r0, sl] - buf[2 * _CHUNK + r0, sl])
          new[_GROUPS + g] = new[_GROUPS + g] + buf[r1, sl] * (
              buf[_CHUNK + r1, sl] - buf[2 * _CHUNK + r1, sl])
        return tuple(new)

      acc = lax.fori_loop(0, _CHUNK // 2, row_body, acc)

    total = acc[0]
    for g in range(1, 2 * _GROUPS):
      total = total + acc[g]
    accv[...] = total
    pltpu.sync_copy(accv, out_hbm.at[wid])

  return partial_kernel


def kernel(target, context, negative_samples, embeddings):
  b = target.shape[0]
  info = plsc.get_sparse_core_info()
  nw = info.num_cores * info.num_subcores
  per_w = b // nw
  chunks = per_w // _CHUNK
  t = target.reshape(nw, chunks, 1, _CHUNK)
  c = context.reshape(nw, chunks, 1, _CHUNK)
  n = negative_samples.reshape(nw, chunks, 1, _CHUNK)
  comb = jnp.concatenate([t, c, n], axis=2).reshape(nw, chunks * 3 * _CHUNK)
  partials = _make_partial_kernel(nw, info.num_cores, chunks)(
      comb, embeddings)
  return -(jnp.sum(partials) / b)


# speedup vs baseline: 1.0133x; 1.0032x over previous
import functools

import jax
import jax.numpy as jnp
from jax import lax
from jax.experimental import pallas as pl
from jax.experimental.pallas import tpu as pltpu
from jax.experimental.pallas import tpu_sc as plsc

_D = 128
_CHUNK = 128
_NBUF = 2
_LANES = 16
_GROUPS = _D // _LANES


@functools.lru_cache(maxsize=None)
def _make_partial_kernel(nw: int, nc: int, chunks: int):
  mesh = plsc.VectorSubcoreMesh(core_axis_name="c", subcore_axis_name="s")

  row_bufs = [pltpu.VMEM((3 * _CHUNK, _D), jnp.float32)
              for _ in range(_NBUF)]

  @functools.partial(
      pl.kernel,
      mesh=mesh,
      out_type=jax.ShapeDtypeStruct((nw, _LANES), jnp.float32),
      scratch_types=[
          pltpu.VMEM((chunks * 3 * _CHUNK,), jnp.int32),
          *row_bufs,
          pltpu.VMEM((_LANES,), jnp.float32),
          *[pltpu.SemaphoreType.DMA for _ in range(_NBUF + 2)],
      ],
  )
  def partial_kernel(idx_hbm, emb_hbm, out_hbm, idx, *rest):
    rows = rest[:_NBUF]
    accv = rest[_NBUF]
    sems = rest[_NBUF + 1:]
    sem_idx0 = sems[_NBUF]
    sem_idx = sems[_NBUF + 1]

    wid = lax.axis_index("s") * nc + lax.axis_index("c")
    per = 3 * _CHUNK
    cp_idx0 = pltpu.async_copy(idx_hbm.at[wid, pl.ds(0, per)],
                               idx.at[pl.ds(0, per)], sem_idx0)
    cp_rest = pltpu.async_copy(
        idx_hbm.at[wid, pl.ds(per, (chunks - 1) * per)],
        idx.at[pl.ds(per, (chunks - 1) * per)], sem_idx)

    def fire(j):
      return pltpu.async_copy(emb_hbm.at[idx.at[pl.ds(3 * _CHUNK * j,
                                                      3 * _CHUNK)]],
                              rows[j % _NBUF], sems[j % _NBUF])

    cp_idx0.wait()
    inflight = {0: fire(0)}
    cp_rest.wait()
    acc = tuple(jnp.zeros((_LANES,), jnp.float32) for _ in range(2 * _GROUPS))
    for j in range(chunks):
      if j + 1 < chunks:
        inflight[j + 1] = fire(j + 1)
      inflight.pop(j).wait()
      buf = rows[j % _NBUF]

      def row_body(i, a, buf=buf):
        r0 = 2 * i
        r1 = r0 + 1
        new = list(a)
        for g in range(_GROUPS):
          sl = pl.ds(g * _LANES, _LANES)
          new[g] = new[g] + buf[r0, sl] * (
              buf[_CHUNK + r0, sl] - buf[2 * _CHUNK + r0, sl])
          new[_GROUPS + g] = new[_GROUPS + g] + buf[r1, sl] * (
              buf[_CHUNK + r1, sl] - buf[2 * _CHUNK + r1, sl])
        return tuple(new)

      acc = lax.fori_loop(0, _CHUNK // 2, row_body, acc)

    total = acc[0]
    for g in range(1, 2 * _GROUPS):
      total = total + acc[g]
    accv[...] = total
    pltpu.sync_copy(accv, out_hbm.at[wid])

  return partial_kernel


def kernel(target, context, negative_samples, embeddings):
  b = target.shape[0]
  info = plsc.get_sparse_core_info()
  nw = info.num_cores * info.num_subcores
  per_w = b // nw
  chunks = per_w // _CHUNK
  t = target.reshape(nw, chunks, 1, _CHUNK)
  c = context.reshape(nw, chunks, 1, _CHUNK)
  n = negative_samples.reshape(nw, chunks, 1, _CHUNK)
  comb = jnp.concatenate([t, c, n], axis=2).reshape(nw, chunks * 3 * _CHUNK)
  partials = _make_partial_kernel(nw, info.num_cores, chunks)(
      comb, embeddings)
  return -(jnp.sum(partials) / b)
